# baseline (speedup 1.0000x reference)
import functools

import jax
import jax.numpy as jnp
import numpy as np
from jax import lax
from jax.experimental import pallas as pl
from jax.experimental.pallas import tpu as pltpu
from jax.experimental.pallas import tpu_sc as plsc

_SIGN = np.uint32(0x80000000)
_MBLK = 512


def _prep_body(uv_ref, scal_ref, cwb_ref, depth_ref, pct_ref,
               at_ref, bt_ref, a2_ref):
    def s(i):
        return scal_ref[0, 0, i]
    fx, fy, cx, cy, sk = s(0), s(1), s(2), s(3), s(4)
    ux = uv_ref[0, :]
    uy = uv_ref[1, :]
    xl = (ux - cx + cy * sk / fy - sk * uy / fy) / fx
    yl = (uy - cy) / fy
    one = jnp.ones_like(ux)
    cam_rel = jnp.concatenate(
        [xl[None, :], yl[None, :], one[None, :], one[None, :]],
        axis=0).astype(jnp.bfloat16)
    wr = lax.dot_general(cwb_ref[0], cam_rel, (((1,), (0,)), ((), ())),
                         preferred_element_type=jnp.float32)
    depth = depth_ref[0, 0, :]
    p = []
    d = []
    for i in range(3):
        d.append(wr[i, :] - s(17 + i))
    nrm = jnp.sqrt((d[0] * d[0] + d[1] * d[1]) + d[2] * d[2])
    for i in range(3):
        p.append(depth * (d[i] / nrm) + s(17 + i))
    a2_ref[0, 0, :] = (p[0] * p[0] + p[1] * p[1]) + p[2] * p[2]
    one = jnp.ones_like(depth)
    zero = jnp.zeros_like(depth)
    at = jnp.concatenate([(-2.0 * p[0])[None, :], (-2.0 * p[1])[None, :],
                          (-2.0 * p[2])[None, :], one[None, :], one[None, :],
                          one[None, :], zero[None, :], zero[None, :]], axis=0)
    at_ref[0] = at.astype(jnp.bfloat16)

    px = pct_ref[0, 0, :]
    py = pct_ref[0, 1, :]
    pz = pct_ref[0, 2, :]
    b2 = (px * px + py * py) + pz * pz
    r1 = b2 - b2.astype(jnp.bfloat16).astype(jnp.float32)
    r2 = r1 - r1.astype(jnp.bfloat16).astype(jnp.float32)
    zb = jnp.zeros_like(b2)
    bt = jnp.concatenate([px[None, :], py[None, :], pz[None, :],
                          b2[None, :], r1[None, :], r2[None, :],
                          zb[None, :], zb[None, :]], axis=0)
    bt_ref[0] = bt.astype(jnp.bfloat16)


def _dist_body(at_ref, bt_ref, a2_ref, out_ref):
    a = at_ref[0]
    b = bt_ref[0]
    acc = lax.dot_general(b, a, (((0,), (0,)), ((), ())),
                          preferred_element_type=jnp.float32)
    out_ref[0, 0, 0] = jnp.min(acc, axis=0) + a2_ref[0, 0, 0]


def _count_lt(keys, cand):
    m = (keys < cand).astype(jnp.int32)
    return jnp.sum(jnp.sum(m, axis=2, keepdims=True), axis=1, keepdims=True)


def _select_body(d_ref, out_ref, *, k, scale):
    x = d_ref[...]
    B = x.shape[0]
    u = lax.bitcast_convert_type(x, jnp.uint32)
    keys = jnp.where(u >= _SIGN, ~u, u | _SIGN)
    t = jnp.zeros((B, 1, 1), jnp.uint32)
    for bit in range(31, -1, -1):
        cand = t | np.uint32(1 << bit)
        cnt = _count_lt(keys, cand)
        t = jnp.where(cnt < k, cand, t)
    cnt_less = _count_lt(keys, t)
    sum_less = jnp.sum(jnp.sum(jnp.where(keys < t, x, 0.0),
                               axis=2, keepdims=True), axis=1, keepdims=True)
    tu = jnp.where(t >= _SIGN, t & np.uint32(0x7FFFFFFF), ~t)
    vt = lax.bitcast_convert_type(tu, jnp.float32)
    total = sum_less + (k - cnt_less).astype(jnp.float32) * vt
    out_ref[...] = jnp.broadcast_to(total * scale, (B, 1, 128))


def _sc_select_call(distr2, B, M, k, scale):
    nv = M // 16
    mesh = plsc.VectorSubcoreMesh(core_axis_name="c", subcore_axis_name="s")

    def body(dist_hbm, out_hbm, xv, kv, ov, fsum_ref):
        wid = lax.axis_index("s") * 2 + lax.axis_index("c")

        def lane_sum(x, ref):
            lane = lax.iota(jnp.int32, 16)
            for sh in (1, 2, 4, 8):
                ref[...] = x
                x = x + plsc.load_gather(ref, [jnp.bitwise_xor(lane, sh)])
            return x

        @pl.when(wid < B)
        def _():
            pltpu.sync_copy(dist_hbm.at[wid], xv)

            def keys_body(i, _):
                x = xv[pl.ds(i * 16, 16)]
                u = lax.bitcast_convert_type(x, jnp.uint32)
                kv[pl.ds(i * 16, 16)] = jnp.where(u >= _SIGN, ~u, u | _SIGN)
                return 0

            lax.fori_loop(0, nv, keys_body, 0)
            t = jnp.zeros((16,), jnp.uint32)
            for bit in range(31, -1, -1):
                cand = t | np.uint32(1 << bit)

                def cnt_body(i, c):
                    m = kv[pl.ds(i * 16, 16)] < cand
                    return c + jnp.where(m, 1.0, 0.0)

                cvec = lax.fori_loop(0, nv, cnt_body,
                                     jnp.zeros((16,), jnp.float32))
                cnt = lane_sum(cvec, fsum_ref)
                t = jnp.where(cnt < k, cand, t)

            def sum_body(i, carry):
                c, sv = carry
                m = kv[pl.ds(i * 16, 16)] < t
                c = c + jnp.where(m, 1.0, 0.0)
                sv = sv + jnp.where(m, xv[pl.ds(i * 16, 16)], 0.0)
                return c, sv

            cvec, svec = lax.fori_loop(
                0, nv, sum_body,
                (jnp.zeros((16,), jnp.float32), jnp.zeros((16,), jnp.float32)))
            cnt_less = lane_sum(cvec, fsum_ref)
            sum_less = lane_sum(svec, fsum_ref)
            tu = jnp.where(t >= _SIGN, t & np.uint32(0x7FFFFFFF), ~t)
            vt = lax.bitcast_convert_type(tu, jnp.float32)
            ov[...] = (sum_less + (k - cnt_less) * vt) * scale
            pltpu.sync_copy(ov, out_hbm.at[wid])

    import functools as _ft
    fn = _ft.partial(
        pl.kernel, mesh=mesh,
        out_type=jax.ShapeDtypeStruct((B, 16), jnp.float32),
        scratch_types=[pltpu.VMEM((M,), jnp.float32),
                       pltpu.VMEM((M,), jnp.uint32),
                       pltpu.VMEM((16,), jnp.float32),
                       pltpu.VMEM((16,), jnp.float32)],
        compiler_params=pltpu.CompilerParams(needs_layout_passes=False),
    )(body)
    return fn(distr2)


def _uv_rows(res):
    g = np.arange(res, dtype=np.float32)
    uv = (np.stack(np.meshgrid(g, g, indexing='ij'))
          * np.float32(1.0 / res) + np.float32(0.5 / res))
    uv = np.flip(uv, axis=0).reshape(2, -1)
    return np.ascontiguousarray(uv)


def kernel(c, image, image_depth, pc, neural_rendering_resolution):
    B = c.shape[0]
    res = image.shape[-1]
    M = res * res
    pc3 = pc[..., :3]
    N = pc3.shape[1]
    half = min(M, N) // 2
    mb = M // _MBLK

    cam2world = c[:, :16].reshape(-1, 4, 4)
    intr = c[:, 16:25].reshape(-1, 3, 3)
    cw = cam2world.astype(jnp.bfloat16).astype(jnp.float32)
    scal = jnp.concatenate([
        intr[:, 0, 0:1], intr[:, 1, 1:2], intr[:, 0, 2:3], intr[:, 1, 2:3],
        intr[:, 0, 1:2], cw[:, :3, :].reshape(B, 12),
        cam2world[:, :3, 3]], axis=1).reshape(B, 1, 20)
    uv = jnp.asarray(_uv_rows(res))
    cwb = cam2world.astype(jnp.bfloat16)[:, :3, :]
    depth3 = image_depth.reshape(B, 1, M)
    pct = pc3.transpose(0, 2, 1)

    at, bt, a2o = pl.pallas_call(
        _prep_body,
        grid=(B,),
        in_specs=[
            pl.BlockSpec((2, M), lambda b: (0, 0)),
            pl.BlockSpec((1, 1, 20), lambda b: (b, 0, 0),
                         memory_space=pltpu.SMEM),
            pl.BlockSpec((1, 3, 4), lambda b: (b, 0, 0)),
            pl.BlockSpec((1, 1, M), lambda b: (b, 0, 0)),
            pl.BlockSpec((1, 3, N), lambda b: (b, 0, 0)),
        ],
        out_specs=[
            pl.BlockSpec((1, 8, M), lambda b: (b, 0, 0)),
            pl.BlockSpec((1, 8, N), lambda b: (b, 0, 0)),
            pl.BlockSpec((1, 1, M), lambda b: (b, 0, 0)),
        ],
        out_shape=[
            jax.ShapeDtypeStruct((B, 8, M), jnp.bfloat16),
            jax.ShapeDtypeStruct((B, 8, N), jnp.bfloat16),
            jax.ShapeDtypeStruct((B, 1, M), jnp.float32),
        ],
    )(uv, scal, cwb, depth3, pct)

    a2r = a2o.reshape(B, mb, 1, _MBLK)
    dist = pl.pallas_call(
        _dist_body,
        grid=(B, mb),
        in_specs=[
            pl.BlockSpec((1, 8, _MBLK), lambda b, m: (b, 0, m)),
            pl.BlockSpec((1, 8, N), lambda b, m: (b, 0, 0)),
            pl.BlockSpec((1, 1, 1, _MBLK), lambda b, m: (b, m, 0, 0)),
        ],
        out_specs=pl.BlockSpec((1, 1, 1, _MBLK), lambda b, m: (b, m, 0, 0)),
        out_shape=jax.ShapeDtypeStruct((B, mb, 1, _MBLK), jnp.float32),
    )(at, bt, a2r)

    distr2 = dist.reshape(B, M)
    sel = _sc_select_call(distr2, B, M, half, 2.0 / half)

    res_t = jnp.asarray(neural_rendering_resolution)
    res_unit = (res_t // res_t).astype(jnp.float32)
    return sel[:, :1] * res_unit


# speedup vs baseline: 1.7378x; 1.7378x over previous
import functools

import jax
import jax.numpy as jnp
import numpy as np
from jax import lax
from jax.experimental import pallas as pl
from jax.experimental.pallas import tpu as pltpu
from jax.experimental.pallas import tpu_sc as plsc

_SIGN = np.uint32(0x80000000)
_MBLK = 512


def _prep_into(uv_ref, scal_ref, cwb_ref, depth_ref, pct_ref,
               at_ref, bt_ref, a2_ref):
    def s(i):
        return scal_ref[0, 0, i]
    fx, fy, cx, cy, sk = s(0), s(1), s(2), s(3), s(4)
    ux = uv_ref[0, :]
    uy = uv_ref[1, :]
    xl = (ux - cx + cy * sk / fy - sk * uy / fy) / fx
    yl = (uy - cy) / fy
    one = jnp.ones_like(ux)
    cam_rel = jnp.concatenate(
        [xl[None, :], yl[None, :], one[None, :], one[None, :]],
        axis=0).astype(jnp.bfloat16)
    wr = lax.dot_general(cwb_ref[0], cam_rel, (((1,), (0,)), ((), ())),
                         preferred_element_type=jnp.float32)
    depth = depth_ref[0, 0, :]
    p = []
    d = []
    for i in range(3):
        d.append(wr[i, :] - s(17 + i))
    nrm = jnp.sqrt((d[0] * d[0] + d[1] * d[1]) + d[2] * d[2])
    for i in range(3):
        p.append(depth * (d[i] / nrm) + s(17 + i))
    a2_ref[0, :] = (p[0] * p[0] + p[1] * p[1]) + p[2] * p[2]
    one = jnp.ones_like(depth)
    zero = jnp.zeros_like(depth)
    at = jnp.concatenate([(-2.0 * p[0])[None, :], (-2.0 * p[1])[None, :],
                          (-2.0 * p[2])[None, :], one[None, :], one[None, :],
                          one[None, :], zero[None, :], zero[None, :]], axis=0)
    at_ref[...] = at.astype(jnp.bfloat16)

    px = pct_ref[0, 0, :]
    py = pct_ref[0, 1, :]
    pz = pct_ref[0, 2, :]
    b2 = (px * px + py * py) + pz * pz
    r1 = b2 - b2.astype(jnp.bfloat16).astype(jnp.float32)
    r2 = r1 - r1.astype(jnp.bfloat16).astype(jnp.float32)
    zb = jnp.zeros_like(b2)
    bt = jnp.concatenate([px[None, :], py[None, :], pz[None, :],
                          b2[None, :], r1[None, :], r2[None, :],
                          zb[None, :], zb[None, :]], axis=0)
    bt_ref[...] = bt.astype(jnp.bfloat16)


def _prep_dist_body(uv_ref, scal_ref, cwb_ref, depth_ref, pct_ref,
                    out_ref, at_s, bt_s, a2_s):
    m = pl.program_id(1)

    @pl.when(m == 0)
    def _prep():
        _prep_into(uv_ref, scal_ref, cwb_ref, depth_ref, pct_ref,
                   at_s, bt_s, a2_s)

    mm = pl.multiple_of(m * _MBLK, _MBLK)
    a = at_s[:, pl.ds(mm, _MBLK)]
    b = bt_s[...]
    acc = lax.dot_general(b, a, (((0,), (0,)), ((), ())),
                          preferred_element_type=jnp.float32)
    out_ref[0, 0, 0] = jnp.min(acc, axis=0) + a2_s[0, pl.ds(mm, _MBLK)]


def _count_lt(keys, cand):
    m = (keys < cand).astype(jnp.int32)
    return jnp.sum(jnp.sum(m, axis=2, keepdims=True), axis=1, keepdims=True)


def _select_body(d_ref, out_ref, *, k, scale):
    x = d_ref[...]
    B = x.shape[0]
    u = lax.bitcast_convert_type(x, jnp.uint32)
    keys = jnp.where(u >= _SIGN, ~u, u | _SIGN)
    t = jnp.zeros((B, 1, 1), jnp.uint32)
    for bit in range(31, -1, -1):
        cand = t | np.uint32(1 << bit)
        cnt = _count_lt(keys, cand)
        t = jnp.where(cnt < k, cand, t)
    cnt_less = _count_lt(keys, t)
    sum_less = jnp.sum(jnp.sum(jnp.where(keys < t, x, 0.0),
                               axis=2, keepdims=True), axis=1, keepdims=True)
    tu = jnp.where(t >= _SIGN, t & np.uint32(0x7FFFFFFF), ~t)
    vt = lax.bitcast_convert_type(tu, jnp.float32)
    total = sum_less + (k - cnt_less).astype(jnp.float32) * vt
    out_ref[...] = jnp.broadcast_to(total * scale, (B, 1, 128))


def _sc_select_call(distr2, B, M, k, scale):
    nv = M // 16
    mesh = plsc.VectorSubcoreMesh(core_axis_name="c", subcore_axis_name="s")

    def body(dist_hbm, out_hbm, xv, kv, ov, fsum_ref):
        wid = lax.axis_index("s") * 2 + lax.axis_index("c")

        def lane_sum(x, ref):
            lane = lax.iota(jnp.int32, 16)
            for sh in (1, 2, 4, 8):
                ref[...] = x
                x = x + plsc.load_gather(ref, [jnp.bitwise_xor(lane, sh)])
            return x

        @pl.when(wid < B)
        def _():
            pltpu.sync_copy(dist_hbm.at[wid], xv)

            def keys_body(i, _):
                x = xv[pl.ds(i * 16, 16)]
                u = lax.bitcast_convert_type(x, jnp.uint32)
                kv[pl.ds(i * 16, 16)] = jnp.where(u >= _SIGN, ~u, u | _SIGN)
                return 0

            lax.fori_loop(0, nv, keys_body, 0)
            t = jnp.zeros((16,), jnp.uint32)
            for bit in range(31, -1, -1):
                cand = t | np.uint32(1 << bit)

                def cnt_body(i, c):
                    m = kv[pl.ds(i * 16, 16)] < cand
                    return c + jnp.where(m, 1.0, 0.0)

                cvec = lax.fori_loop(0, nv, cnt_body,
                                     jnp.zeros((16,), jnp.float32))
                cnt = lane_sum(cvec, fsum_ref)
                t = jnp.where(cnt < k, cand, t)

            def sum_body(i, carry):
                c, sv = carry
                m = kv[pl.ds(i * 16, 16)] < t
                c = c + jnp.where(m, 1.0, 0.0)
                sv = sv + jnp.where(m, xv[pl.ds(i * 16, 16)], 0.0)
                return c, sv

            cvec, svec = lax.fori_loop(
                0, nv, sum_body,
                (jnp.zeros((16,), jnp.float32), jnp.zeros((16,), jnp.float32)))
            cnt_less = lane_sum(cvec, fsum_ref)
            sum_less = lane_sum(svec, fsum_ref)
            tu = jnp.where(t >= _SIGN, t & np.uint32(0x7FFFFFFF), ~t)
            vt = lax.bitcast_convert_type(tu, jnp.float32)
            ov[...] = (sum_less + (k - cnt_less) * vt) * scale
            pltpu.sync_copy(ov, out_hbm.at[wid])

    import functools as _ft
    fn = _ft.partial(
        pl.kernel, mesh=mesh,
        out_type=jax.ShapeDtypeStruct((B, 16), jnp.float32),
        scratch_types=[pltpu.VMEM((M,), jnp.float32),
                       pltpu.VMEM((M,), jnp.uint32),
                       pltpu.VMEM((16,), jnp.float32),
                       pltpu.VMEM((16,), jnp.float32)],
        compiler_params=pltpu.CompilerParams(needs_layout_passes=False),
    )(body)
    return fn(distr2)


def _uv_rows(res):
    g = np.arange(res, dtype=np.float32)
    uv = (np.stack(np.meshgrid(g, g, indexing='ij'))
          * np.float32(1.0 / res) + np.float32(0.5 / res))
    uv = np.flip(uv, axis=0).reshape(2, -1)
    return np.ascontiguousarray(uv)


def kernel(c, image, image_depth, pc, neural_rendering_resolution):
    B = c.shape[0]
    res = image.shape[-1]
    M = res * res
    pc3 = pc[..., :3]
    N = pc3.shape[1]
    half = min(M, N) // 2
    mb = M // _MBLK

    cam2world = c[:, :16].reshape(-1, 4, 4)
    intr = c[:, 16:25].reshape(-1, 3, 3)
    cw = cam2world.astype(jnp.bfloat16).astype(jnp.float32)
    scal = jnp.concatenate([
        intr[:, 0, 0:1], intr[:, 1, 1:2], intr[:, 0, 2:3], intr[:, 1, 2:3],
        intr[:, 0, 1:2], cw[:, :3, :].reshape(B, 12),
        cam2world[:, :3, 3]], axis=1).reshape(B, 1, 20)
    uv = jnp.asarray(_uv_rows(res))
    cwb = cam2world.astype(jnp.bfloat16)[:, :3, :]
    depth3 = image_depth.reshape(B, 1, M)
    pct = pc3.transpose(0, 2, 1)

    dist = pl.pallas_call(
        _prep_dist_body,
        grid=(B, mb),
        in_specs=[
            pl.BlockSpec((2, M), lambda b, m: (0, 0)),
            pl.BlockSpec((1, 1, 20), lambda b, m: (b, 0, 0),
                         memory_space=pltpu.SMEM),
            pl.BlockSpec((1, 3, 4), lambda b, m: (b, 0, 0)),
            pl.BlockSpec((1, 1, M), lambda b, m: (b, 0, 0)),
            pl.BlockSpec((1, 3, N), lambda b, m: (b, 0, 0)),
        ],
        out_specs=pl.BlockSpec((1, 1, 1, _MBLK), lambda b, m: (b, m, 0, 0)),
        out_shape=jax.ShapeDtypeStruct((B, mb, 1, _MBLK), jnp.float32),
        scratch_shapes=[
            pltpu.VMEM((8, M), jnp.bfloat16),
            pltpu.VMEM((8, N), jnp.bfloat16),
            pltpu.VMEM((1, M), jnp.float32),
        ],
    )(uv, scal, cwb, depth3, pct)

    distr = dist.reshape(B, mb, _MBLK)
    sel = pl.pallas_call(
        functools.partial(_select_body, k=half, scale=2.0 / half),
        in_specs=[pl.BlockSpec((B, mb, _MBLK), lambda: (0, 0, 0))],
        out_specs=pl.BlockSpec((B, 1, 128), lambda: (0, 0, 0)),
        out_shape=jax.ShapeDtypeStruct((B, 1, 128), jnp.float32),
    )(distr)

    res_t = jnp.asarray(neural_rendering_resolution)
    res_unit = (res_t // res_t).astype(jnp.float32)
    return sel[:, 0, :1] * res_unit


# speedup vs baseline: 1.8452x; 1.0618x over previous
import functools

import jax
import jax.numpy as jnp
import numpy as np
from jax import lax
from jax.experimental import pallas as pl
from jax.experimental.pallas import tpu as pltpu
from jax.experimental.pallas import tpu_sc as plsc

_SIGN = np.uint32(0x80000000)
_MBLK = 1024


def _prep_into(uv_ref, scal_ref, cwb_ref, depth_ref, pct_ref,
               at_ref, bt_ref, a2_ref):
    def s(i):
        return scal_ref[0, 0, i]
    fx, fy, cx, cy, sk = s(0), s(1), s(2), s(3), s(4)
    ux = uv_ref[0, :]
    uy = uv_ref[1, :]
    xl = (ux - cx + cy * sk / fy - sk * uy / fy) / fx
    yl = (uy - cy) / fy
    one = jnp.ones_like(ux)
    cam_rel = jnp.concatenate(
        [xl[None, :], yl[None, :], one[None, :], one[None, :]],
        axis=0).astype(jnp.bfloat16)
    wr = lax.dot_general(cwb_ref[0], cam_rel, (((1,), (0,)), ((), ())),
                         preferred_element_type=jnp.float32)
    depth = depth_ref[0, 0, :]
    p = []
    d = []
    for i in range(3):
        d.append(wr[i, :] - s(17 + i))
    nrm = jnp.sqrt((d[0] * d[0] + d[1] * d[1]) + d[2] * d[2])
    for i in range(3):
        p.append(depth * (d[i] / nrm) + s(17 + i))
    a2_ref[0, :] = (p[0] * p[0] + p[1] * p[1]) + p[2] * p[2]
    one = jnp.ones_like(depth)
    zero = jnp.zeros_like(depth)
    at = jnp.concatenate([(-2.0 * p[0])[None, :], (-2.0 * p[1])[None, :],
                          (-2.0 * p[2])[None, :], one[None, :], one[None, :],
                          one[None, :], zero[None, :], zero[None, :]], axis=0)
    at_ref[...] = at.astype(jnp.bfloat16)

    px = pct_ref[0, 0, :]
    py = pct_ref[0, 1, :]
    pz = pct_ref[0, 2, :]
    b2 = (px * px + py * py) + pz * pz
    r1 = b2 - b2.astype(jnp.bfloat16).astype(jnp.float32)
    r2 = r1 - r1.astype(jnp.bfloat16).astype(jnp.float32)
    zb = jnp.zeros_like(b2)
    bt = jnp.concatenate([px[None, :], py[None, :], pz[None, :],
                          b2[None, :], r1[None, :], r2[None, :],
                          zb[None, :], zb[None, :]], axis=0)
    bt_ref[...] = bt.astype(jnp.bfloat16)


def _prep_dist_body(uv_ref, scal_ref, cwb_ref, depth_ref, pct_ref,
                    out_ref, at_s, bt_s, a2_s):
    m = pl.program_id(1)

    @pl.when(m == 0)
    def _prep():
        _prep_into(uv_ref, scal_ref, cwb_ref, depth_ref, pct_ref,
                   at_s, bt_s, a2_s)

    mm = pl.multiple_of(m * _MBLK, _MBLK)
    a = at_s[:, pl.ds(mm, _MBLK)]
    b = bt_s[...]
    acc = lax.dot_general(b, a, (((0,), (0,)), ((), ())),
                          preferred_element_type=jnp.float32)
    out_ref[0, 0, 0] = jnp.min(acc, axis=0) + a2_s[0, pl.ds(mm, _MBLK)]


def _count_lt(keys, cand):
    m = (keys < cand).astype(jnp.int32)
    return jnp.sum(jnp.sum(m, axis=2, keepdims=True), axis=1, keepdims=True)


def _select_body(d_ref, out_ref, *, k, scale):
    x = d_ref[...]
    B = x.shape[0]
    u = lax.bitcast_convert_type(x, jnp.uint32)
    keys = jnp.where(u >= _SIGN, ~u, u | _SIGN)
    t = jnp.zeros((B, 1, 1), jnp.uint32)
    for bit in range(31, -1, -1):
        cand = t | np.uint32(1 << bit)
        cnt = _count_lt(keys, cand)
        t = jnp.where(cnt < k, cand, t)
    cnt_less = _count_lt(keys, t)
    sum_less = jnp.sum(jnp.sum(jnp.where(keys < t, x, 0.0),
                               axis=2, keepdims=True), axis=1, keepdims=True)
    tu = jnp.where(t >= _SIGN, t & np.uint32(0x7FFFFFFF), ~t)
    vt = lax.bitcast_convert_type(tu, jnp.float32)
    total = sum_less + (k - cnt_less).astype(jnp.float32) * vt
    out_ref[...] = jnp.broadcast_to(total * scale, (B, 1, 128))


def _sc_select_call(distr2, B, M, k, scale):
    nv = M // 16
    mesh = plsc.VectorSubcoreMesh(core_axis_name="c", subcore_axis_name="s")

    def body(dist_hbm, out_hbm, xv, kv, ov, fsum_ref):
        wid = lax.axis_index("s") * 2 + lax.axis_index("c")

        def lane_sum(x, ref):
            lane = lax.iota(jnp.int32, 16)
            for sh in (1, 2, 4, 8):
                ref[...] = x
                x = x + plsc.load_gather(ref, [jnp.bitwise_xor(lane, sh)])
            return x

        @pl.when(wid < B)
        def _():
            pltpu.sync_copy(dist_hbm.at[wid], xv)

            def keys_body(i, _):
                x = xv[pl.ds(i * 16, 16)]
                u = lax.bitcast_convert_type(x, jnp.uint32)
                kv[pl.ds(i * 16, 16)] = jnp.where(u >= _SIGN, ~u, u | _SIGN)
                return 0

            lax.fori_loop(0, nv, keys_body, 0)
            t = jnp.zeros((16,), jnp.uint32)
            for bit in range(31, -1, -1):
                cand = t | np.uint32(1 << bit)

                def cnt_body(i, c):
                    m = kv[pl.ds(i * 16, 16)] < cand
                    return c + jnp.where(m, 1.0, 0.0)

                cvec = lax.fori_loop(0, nv, cnt_body,
                                     jnp.zeros((16,), jnp.float32))
                cnt = lane_sum(cvec, fsum_ref)
                t = jnp.where(cnt < k, cand, t)

            def sum_body(i, carry):
                c, sv = carry
                m = kv[pl.ds(i * 16, 16)] < t
                c = c + jnp.where(m, 1.0, 0.0)
                sv = sv + jnp.where(m, xv[pl.ds(i * 16, 16)], 0.0)
                return c, sv

            cvec, svec = lax.fori_loop(
                0, nv, sum_body,
                (jnp.zeros((16,), jnp.float32), jnp.zeros((16,), jnp.float32)))
            cnt_less = lane_sum(cvec, fsum_ref)
            sum_less = lane_sum(svec, fsum_ref)
            tu = jnp.where(t >= _SIGN, t & np.uint32(0x7FFFFFFF), ~t)
            vt = lax.bitcast_convert_type(tu, jnp.float32)
            ov[...] = (sum_less + (k - cnt_less) * vt) * scale
            pltpu.sync_copy(ov, out_hbm.at[wid])

    import functools as _ft
    fn = _ft.partial(
        pl.kernel, mesh=mesh,
        out_type=jax.ShapeDtypeStruct((B, 16), jnp.float32),
        scratch_types=[pltpu.VMEM((M,), jnp.float32),
                       pltpu.VMEM((M,), jnp.uint32),
                       pltpu.VMEM((16,), jnp.float32),
                       pltpu.VMEM((16,), jnp.float32)],
        compiler_params=pltpu.CompilerParams(needs_layout_passes=False),
    )(body)
    return fn(distr2)


def _uv_rows(res):
    g = np.arange(res, dtype=np.float32)
    uv = (np.stack(np.meshgrid(g, g, indexing='ij'))
          * np.float32(1.0 / res) + np.float32(0.5 / res))
    uv = np.flip(uv, axis=0).reshape(2, -1)
    return np.ascontiguousarray(uv)


def kernel(c, image, image_depth, pc, neural_rendering_resolution):
    B = c.shape[0]
    res = image.shape[-1]
    M = res * res
    pc3 = pc[..., :3]
    N = pc3.shape[1]
    half = min(M, N) // 2
    mb = M // _MBLK

    cam2world = c[:, :16].reshape(-1, 4, 4)
    intr = c[:, 16:25].reshape(-1, 3, 3)
    cw = cam2world.astype(jnp.bfloat16).astype(jnp.float32)
    scal = jnp.concatenate([
        intr[:, 0, 0:1], intr[:, 1, 1:2], intr[:, 0, 2:3], intr[:, 1, 2:3],
        intr[:, 0, 1:2], cw[:, :3, :].reshape(B, 12),
        cam2world[:, :3, 3]], axis=1).reshape(B, 1, 20)
    uv = jnp.asarray(_uv_rows(res))
    cwb = cam2world.astype(jnp.bfloat16)[:, :3, :]
    depth3 = image_depth.reshape(B, 1, M)
    pct = pc3.transpose(0, 2, 1)

    dist = pl.pallas_call(
        _prep_dist_body,
        grid=(B, mb),
        in_specs=[
            pl.BlockSpec((2, M), lambda b, m: (0, 0)),
            pl.BlockSpec((1, 1, 20), lambda b, m: (b, 0, 0),
                         memory_space=pltpu.SMEM),
            pl.BlockSpec((1, 3, 4), lambda b, m: (b, 0, 0)),
            pl.BlockSpec((1, 1, M), lambda b, m: (b, 0, 0)),
            pl.BlockSpec((1, 3, N), lambda b, m: (b, 0, 0)),
        ],
        out_specs=pl.BlockSpec((1, 1, 1, _MBLK), lambda b, m: (b, m, 0, 0)),
        out_shape=jax.ShapeDtypeStruct((B, mb, 1, _MBLK), jnp.float32),
        scratch_shapes=[
            pltpu.VMEM((8, M), jnp.bfloat16),
            pltpu.VMEM((8, N), jnp.bfloat16),
            pltpu.VMEM((1, M), jnp.float32),
        ],
    )(uv, scal, cwb, depth3, pct)

    distr = dist.reshape(B, mb, _MBLK)
    sel = pl.pallas_call(
        functools.partial(_select_body, k=half, scale=2.0 / half),
        in_specs=[pl.BlockSpec((B, mb, _MBLK), lambda: (0, 0, 0))],
        out_specs=pl.BlockSpec((B, 1, 128), lambda: (0, 0, 0)),
        out_shape=jax.ShapeDtypeStruct((B, 1, 128), jnp.float32),
    )(distr)

    res_t = jnp.asarray(neural_rendering_resolution)
    res_unit = (res_t // res_t).astype(jnp.float32)
    return sel[:, 0, :1] * res_unit


# speedup vs baseline: 1.9083x; 1.0342x over previous
import functools

import jax
import jax.numpy as jnp
import numpy as np
from jax import lax
from jax.experimental import pallas as pl
from jax.experimental.pallas import tpu as pltpu
from jax.experimental.pallas import tpu_sc as plsc

_SIGN = np.uint32(0x80000000)
_MBLK = 2048


def _prep_into(uv_ref, scal_ref, cwb_ref, depth_ref, pct_ref,
               at_ref, bt_ref, a2_ref):
    def s(i):
        return scal_ref[0, 0, i]
    fx, fy, cx, cy, sk = s(0), s(1), s(2), s(3), s(4)
    ux = uv_ref[0, :]
    uy = uv_ref[1, :]
    xl = (ux - cx + cy * sk / fy - sk * uy / fy) / fx
    yl = (uy - cy) / fy
    one = jnp.ones_like(ux)
    cam_rel = jnp.concatenate(
        [xl[None, :], yl[None, :], one[None, :], one[None, :]],
        axis=0).astype(jnp.bfloat16)
    wr = lax.dot_general(cwb_ref[0], cam_rel, (((1,), (0,)), ((), ())),
                         preferred_element_type=jnp.float32)
    depth = depth_ref[0, 0, :]
    p = []
    d = []
    for i in range(3):
        d.append(wr[i, :] - s(17 + i))
    nrm = jnp.sqrt((d[0] * d[0] + d[1] * d[1]) + d[2] * d[2])
    for i in range(3):
        p.append(depth * (d[i] / nrm) + s(17 + i))
    a2_ref[0, :] = (p[0] * p[0] + p[1] * p[1]) + p[2] * p[2]
    one = jnp.ones_like(depth)
    zero = jnp.zeros_like(depth)
    at = jnp.concatenate([(-2.0 * p[0])[None, :], (-2.0 * p[1])[None, :],
                          (-2.0 * p[2])[None, :], one[None, :], one[None, :],
                          one[None, :], zero[None, :], zero[None, :]], axis=0)
    at_ref[...] = at.astype(jnp.bfloat16)

    px = pct_ref[0, 0, :]
    py = pct_ref[0, 1, :]
    pz = pct_ref[0, 2, :]
    b2 = (px * px + py * py) + pz * pz
    r1 = b2 - b2.astype(jnp.bfloat16).astype(jnp.float32)
    r2 = r1 - r1.astype(jnp.bfloat16).astype(jnp.float32)
    zb = jnp.zeros_like(b2)
    bt = jnp.concatenate([px[None, :], py[None, :], pz[None, :],
                          b2[None, :], r1[None, :], r2[None, :],
                          zb[None, :], zb[None, :]], axis=0)
    bt_ref[...] = bt.astype(jnp.bfloat16)


def _prep_dist_body(uv_ref, scal_ref, cwb_ref, depth_ref, pct_ref,
                    out_ref, at_s, bt_s, a2_s):
    m = pl.program_id(1)

    @pl.when(m == 0)
    def _prep():
        _prep_into(uv_ref, scal_ref, cwb_ref, depth_ref, pct_ref,
                   at_s, bt_s, a2_s)

    mm = pl.multiple_of(m * _MBLK, _MBLK)
    a = at_s[:, pl.ds(mm, _MBLK)]
    b = bt_s[...]
    acc = lax.dot_general(b, a, (((0,), (0,)), ((), ())),
                          preferred_element_type=jnp.float32)
    out_ref[0, 0, 0] = jnp.min(acc, axis=0) + a2_s[0, pl.ds(mm, _MBLK)]


def _count_lt(keys, cand):
    m = (keys < cand).astype(jnp.int32)
    return jnp.sum(jnp.sum(m, axis=2, keepdims=True), axis=1, keepdims=True)


def _select_body(d_ref, out_ref, *, k, scale):
    x = d_ref[...]
    B = x.shape[0]
    u = lax.bitcast_convert_type(x, jnp.uint32)
    keys = jnp.where(u >= _SIGN, ~u, u | _SIGN)
    t = jnp.zeros((B, 1, 1), jnp.uint32)
    for bit in range(31, -1, -1):
        cand = t | np.uint32(1 << bit)
        cnt = _count_lt(keys, cand)
        t = jnp.where(cnt < k, cand, t)
    cnt_less = _count_lt(keys, t)
    sum_less = jnp.sum(jnp.sum(jnp.where(keys < t, x, 0.0),
                               axis=2, keepdims=True), axis=1, keepdims=True)
    tu = jnp.where(t >= _SIGN, t & np.uint32(0x7FFFFFFF), ~t)
    vt = lax.bitcast_convert_type(tu, jnp.float32)
    total = sum_less + (k - cnt_less).astype(jnp.float32) * vt
    out_ref[...] = jnp.broadcast_to(total * scale, (B, 1, 128))


def _sc_select_call(distr2, B, M, k, scale):
    nv = M // 16
    mesh = plsc.VectorSubcoreMesh(core_axis_name="c", subcore_axis_name="s")

    def body(dist_hbm, out_hbm, xv, kv, ov, fsum_ref):
        wid = lax.axis_index("s") * 2 + lax.axis_index("c")

        def lane_sum(x, ref):
            lane = lax.iota(jnp.int32, 16)
            for sh in (1, 2, 4, 8):
                ref[...] = x
                x = x + plsc.load_gather(ref, [jnp.bitwise_xor(lane, sh)])
            return x

        @pl.when(wid < B)
        def _():
            pltpu.sync_copy(dist_hbm.at[wid], xv)

            def keys_body(i, _):
                x = xv[pl.ds(i * 16, 16)]
                u = lax.bitcast_convert_type(x, jnp.uint32)
                kv[pl.ds(i * 16, 16)] = jnp.where(u >= _SIGN, ~u, u | _SIGN)
                return 0

            lax.fori_loop(0, nv, keys_body, 0)
            t = jnp.zeros((16,), jnp.uint32)
            for bit in range(31, -1, -1):
                cand = t | np.uint32(1 << bit)

                def cnt_body(i, c):
                    m = kv[pl.ds(i * 16, 16)] < cand
                    return c + jnp.where(m, 1.0, 0.0)

                cvec = lax.fori_loop(0, nv, cnt_body,
                                     jnp.zeros((16,), jnp.float32))
                cnt = lane_sum(cvec, fsum_ref)
                t = jnp.where(cnt < k, cand, t)

            def sum_body(i, carry):
                c, sv = carry
                m = kv[pl.ds(i * 16, 16)] < t
                c = c + jnp.where(m, 1.0, 0.0)
                sv = sv + jnp.where(m, xv[pl.ds(i * 16, 16)], 0.0)
                return c, sv

            cvec, svec = lax.fori_loop(
                0, nv, sum_body,
                (jnp.zeros((16,), jnp.float32), jnp.zeros((16,), jnp.float32)))
            cnt_less = lane_sum(cvec, fsum_ref)
            sum_less = lane_sum(svec, fsum_ref)
            tu = jnp.where(t >= _SIGN, t & np.uint32(0x7FFFFFFF), ~t)
            vt = lax.bitcast_convert_type(tu, jnp.float32)
            ov[...] = (sum_less + (k - cnt_less) * vt) * scale
            pltpu.sync_copy(ov, out_hbm.at[wid])

    import functools as _ft
    fn = _ft.partial(
        pl.kernel, mesh=mesh,
        out_type=jax.ShapeDtypeStruct((B, 16), jnp.float32),
        scratch_types=[pltpu.VMEM((M,), jnp.float32),
                       pltpu.VMEM((M,), jnp.uint32),
                       pltpu.VMEM((16,), jnp.float32),
                       pltpu.VMEM((16,), jnp.float32)],
        compiler_params=pltpu.CompilerParams(needs_layout_passes=False),
    )(body)
    return fn(distr2)


def _uv_rows(res):
    g = np.arange(res, dtype=np.float32)
    uv = (np.stack(np.meshgrid(g, g, indexing='ij'))
          * np.float32(1.0 / res) + np.float32(0.5 / res))
    uv = np.flip(uv, axis=0).reshape(2, -1)
    return np.ascontiguousarray(uv)


def kernel(c, image, image_depth, pc, neural_rendering_resolution):
    B = c.shape[0]
    res = image.shape[-1]
    M = res * res
    pc3 = pc[..., :3]
    N = pc3.shape[1]
    half = min(M, N) // 2
    mb = M // _MBLK

    cam2world = c[:, :16].reshape(-1, 4, 4)
    intr = c[:, 16:25].reshape(-1, 3, 3)
    cw = cam2world.astype(jnp.bfloat16).astype(jnp.float32)
    scal = jnp.concatenate([
        intr[:, 0, 0:1], intr[:, 1, 1:2], intr[:, 0, 2:3], intr[:, 1, 2:3],
        intr[:, 0, 1:2], cw[:, :3, :].reshape(B, 12),
        cam2world[:, :3, 3]], axis=1).reshape(B, 1, 20)
    uv = jnp.asarray(_uv_rows(res))
    cwb = cam2world.astype(jnp.bfloat16)[:, :3, :]
    depth3 = image_depth.reshape(B, 1, M)
    pct = pc3.transpose(0, 2, 1)

    dist = pl.pallas_call(
        _prep_dist_body,
        grid=(B, mb),
        in_specs=[
            pl.BlockSpec((2, M), lambda b, m: (0, 0)),
            pl.BlockSpec((1, 1, 20), lambda b, m: (b, 0, 0),
                         memory_space=pltpu.SMEM),
            pl.BlockSpec((1, 3, 4), lambda b, m: (b, 0, 0)),
            pl.BlockSpec((1, 1, M), lambda b, m: (b, 0, 0)),
            pl.BlockSpec((1, 3, N), lambda b, m: (b, 0, 0)),
        ],
        out_specs=pl.BlockSpec((1, 1, 1, _MBLK), lambda b, m: (b, m, 0, 0)),
        out_shape=jax.ShapeDtypeStruct((B, mb, 1, _MBLK), jnp.float32),
        scratch_shapes=[
            pltpu.VMEM((8, M), jnp.bfloat16),
            pltpu.VMEM((8, N), jnp.bfloat16),
            pltpu.VMEM((1, M), jnp.float32),
        ],
    )(uv, scal, cwb, depth3, pct)

    distr = dist.reshape(B, mb, _MBLK)
    sel = pl.pallas_call(
        functools.partial(_select_body, k=half, scale=2.0 / half),
        in_specs=[pl.BlockSpec((B, mb, _MBLK), lambda: (0, 0, 0))],
        out_specs=pl.BlockSpec((B, 1, 128), lambda: (0, 0, 0)),
        out_shape=jax.ShapeDtypeStruct((B, 1, 128), jnp.float32),
    )(distr)

    res_t = jnp.asarray(neural_rendering_resolution)
    res_unit = (res_t // res_t).astype(jnp.float32)
    return sel[:, 0, :1] * res_unit
